# Initial kernel scaffold; baseline (speedup 1.0000x reference)
import jax
import jax.numpy as jnp
from jax.experimental import pallas as pl


def kernel(indices, table):
    raise NotImplementedError("write your pallas kernel here")


# speedup vs baseline: 3.9469x; 3.9469x over previous
import functools

import jax
import jax.numpy as jnp
from jax import lax
from jax.experimental import pallas as pl
from jax.experimental.pallas import tpu as pltpu
from jax.experimental.pallas import tpu_sc as plsc

_BATCH = 4096
_HIST = 200
_D = 64
_B = _BATCH * _HIST

_NC = 2
_NS = 16
_NW = _NC * _NS
_B_PER_W = _B // _NW

_CHUNK = 512
_NCHUNK = _B_PER_W // _CHUNK


def _gather_kernel(table_hbm, idx_hbm, out_hbm, idx_v, rows_v, sem):
    wid = lax.axis_index("s") * _NC + lax.axis_index("c")
    base = wid * _B_PER_W

    def body(i, carry):
        off = pl.multiple_of(base + i * _CHUNK, _CHUNK)
        pltpu.sync_copy(idx_hbm.at[pl.ds(off, _CHUNK)], idx_v)
        pltpu.async_copy(table_hbm.at[idx_v], rows_v, sem).wait()
        pltpu.sync_copy(rows_v, out_hbm.at[pl.ds(off, _CHUNK)])
        return carry

    lax.fori_loop(0, _NCHUNK, body, 0)


@jax.jit
def _run(indices, table):
    idx_flat = indices.reshape(_B).astype(jnp.int32)
    mesh = plsc.VectorSubcoreMesh(core_axis_name="c", subcore_axis_name="s")
    k = functools.partial(
        pl.kernel,
        out_type=jax.ShapeDtypeStruct((_B, _D), jnp.float32),
        mesh=mesh,
        scratch_types=[
            pltpu.VMEM((_CHUNK,), jnp.int32),
            pltpu.VMEM((_CHUNK, _D), jnp.float32),
            pltpu.SemaphoreType.DMA,
        ],
        compiler_params=pltpu.CompilerParams(use_tc_tiling_on_sc=False),
    )(_gather_kernel)
    out = k(table, idx_flat)
    return out.reshape(_BATCH, _HIST, _D)


def kernel(indices, table):
    return _run(indices, table)


# speedup vs baseline: 4.1301x; 1.0464x over previous
import functools

import jax
import jax.numpy as jnp
from jax import lax
from jax.experimental import pallas as pl
from jax.experimental.pallas import tpu as pltpu
from jax.experimental.pallas import tpu_sc as plsc

_BATCH = 4096
_HIST = 200
_D = 64
_B = _BATCH * _HIST

_NC = 2
_NS = 16
_NW = _NC * _NS
_B_PER_W = _B // _NW

_CHUNK = 512
_NBUF = 2
assert _B_PER_W % (_CHUNK * _NBUF) == 0


def _gather_kernel(table_hbm, idx_hbm, out_hbm, idx_v, rows_v, in_sems, g_sems, out_sems):
    wid = lax.axis_index("s") * _NC + lax.axis_index("c")
    base = wid * _B_PER_W
    ngroup = _B_PER_W // (_CHUNK * _NBUF)

    def group(g, carry):
        goff = base + g * (_CHUNK * _NBUF)

        @pl.when(g > 0)
        def _():
            for b in range(_NBUF):
                pltpu.make_async_copy(
                    rows_v.at[b], out_hbm.at[pl.ds(0, _CHUNK)], out_sems.at[b]
                ).wait()

        idx_handles = []
        for b in range(_NBUF):
            off = pl.multiple_of(goff + b * _CHUNK, _CHUNK)
            idx_handles.append(
                pltpu.async_copy(idx_hbm.at[pl.ds(off, _CHUNK)], idx_v.at[b], in_sems.at[b])
            )
        gather_handles = []
        for b in range(_NBUF):
            idx_handles[b].wait()
            gather_handles.append(
                pltpu.async_copy(table_hbm.at[idx_v.at[b]], rows_v.at[b], g_sems.at[b])
            )
        for b in range(_NBUF):
            off = pl.multiple_of(goff + b * _CHUNK, _CHUNK)
            gather_handles[b].wait()
            pltpu.async_copy(rows_v.at[b], out_hbm.at[pl.ds(off, _CHUNK)], out_sems.at[b])
        return carry

    lax.fori_loop(0, ngroup, group, 0)

    for b in range(_NBUF):
        pltpu.make_async_copy(
            rows_v.at[b], out_hbm.at[pl.ds(0, _CHUNK)], out_sems.at[b]
        ).wait()


@jax.jit
def _run(indices, table):
    idx_flat = indices.reshape(_B).astype(jnp.int32)
    mesh = plsc.VectorSubcoreMesh(core_axis_name="c", subcore_axis_name="s")
    k = functools.partial(
        pl.kernel,
        out_type=jax.ShapeDtypeStruct((_B, _D), jnp.float32),
        mesh=mesh,
        scratch_types=[
            pltpu.VMEM((_NBUF, _CHUNK), jnp.int32),
            pltpu.VMEM((_NBUF, _CHUNK, _D), jnp.float32),
            pltpu.SemaphoreType.DMA((_NBUF,)),
            pltpu.SemaphoreType.DMA((_NBUF,)),
            pltpu.SemaphoreType.DMA((_NBUF,)),
        ],
        compiler_params=pltpu.CompilerParams(use_tc_tiling_on_sc=False),
    )(_gather_kernel)
    out = k(table, idx_flat)
    return out.reshape(_BATCH, _HIST, _D)


def kernel(indices, table):
    return _run(indices, table)
